# Initial kernel scaffold; baseline (speedup 1.0000x reference)
import jax
import jax.numpy as jnp
from jax.experimental import pallas as pl


def kernel(lex_indices, x, W):
    raise NotImplementedError("write your pallas kernel here")


# speedup vs baseline: 6.5542x; 6.5542x over previous
import functools

import jax
import jax.numpy as jnp
from jax import lax
from jax.experimental import pallas as pl
from jax.experimental.pallas import tpu as pltpu
from jax.experimental.pallas import tpu_sc as plsc

_EPS = 1e-7
_LANES = 16


def _sc_scores(idx_flat, w_flat, V, NLEX):
    n = idx_flat.shape[0]
    info = plsc.get_sparse_core_info()
    nw = info.num_cores * info.num_subcores
    chunk = n // nw
    steps = chunk // _LANES
    mesh = plsc.VectorSubcoreMesh(core_axis_name="c", subcore_axis_name="s")

    @functools.partial(
        pl.kernel,
        mesh=mesh,
        out_type=jax.ShapeDtypeStruct((n,), jnp.float32),
        compiler_params=pltpu.CompilerParams(needs_layout_passes=False),
        scratch_types=[
            pltpu.VMEM((V * NLEX,), jnp.float32),
            pltpu.VMEM((V,), jnp.float32),
            pltpu.VMEM((chunk,), jnp.int32),
            pltpu.VMEM((chunk,), jnp.float32),
        ],
    )
    def k(idx_hbm, w_hbm, s_hbm, wfull_v, w_v, idx_v, s_v):
        wid = lax.axis_index("s") * info.num_cores + lax.axis_index("c")
        base = wid * chunk
        pltpu.sync_copy(w_hbm, wfull_v)
        pltpu.sync_copy(idx_hbm.at[pl.ds(base, chunk)], idx_v)
        for v in range(V // _LANES):
            rows = (lax.iota(jnp.int32, _LANES) + v * _LANES) * NLEX
            acc = jnp.zeros((_LANES,), jnp.float32)
            for j in range(NLEX):
                acc = acc + plsc.load_gather(wfull_v, [rows + j])
            w_v[pl.ds(v * _LANES, _LANES)] = acc

        def body(i, carry):
            off = i * _LANES
            idx16 = idx_v[pl.ds(off, _LANES)]
            s_v[pl.ds(off, _LANES)] = plsc.load_gather(w_v, [idx16])
            return carry

        lax.fori_loop(0, steps, body, 0)
        pltpu.sync_copy(s_v, s_hbm.at[pl.ds(base, chunk)])

    return k(idx_flat, w_flat)


def _tc_pool(s, x):
    B, L, D = x.shape
    bblk = 128

    def body(s_ref, x_ref, o_ref):
        e = jnp.exp(jnp.tanh(s_ref[...]))
        denom = jnp.sum(e, axis=1, keepdims=True) + _EPS
        acc = jnp.sum(x_ref[...] * e[:, :, None], axis=1)
        o_ref[...] = acc / denom

    return pl.pallas_call(
        body,
        grid=(B // bblk,),
        in_specs=[
            pl.BlockSpec((bblk, L), lambda i: (i, 0)),
            pl.BlockSpec((bblk, L, D), lambda i: (i, 0, 0)),
        ],
        out_specs=pl.BlockSpec((bblk, D), lambda i: (i, 0)),
        out_shape=jax.ShapeDtypeStruct((B, D), jnp.float32),
    )(s, x)


def kernel(lex_indices, x, W):
    B, L, D = x.shape
    V, NLEX = W.shape
    s = _sc_scores(lex_indices.reshape(-1), W.reshape(-1), V, NLEX)
    return _tc_pool(s.reshape(B, L), x)
